# baseline (speedup 1.0000x reference)
import jax
import jax.numpy as jnp
from jax import lax
from jax.experimental import pallas as pl
from jax.experimental.pallas import tpu as pltpu
from jax.experimental.pallas import tpu_sc as plsc

_D = 64
_B = 16384
_V = 1000000

_info = plsc.get_sparse_core_info()
_NC, _NS = _info.num_cores, _info.num_subcores
_NW = _NC * _NS
_BPW = _B // _NW
_C = 128
_NCH = _BPW // _C
_HI_MASK = jnp.uint32(0xFFFF0000)


def _gather_body(idx_hbm, tview_hbm, out_hbm, idx_v, q_v, rows2_v, sel_v, sem):
    wid = lax.axis_index("s") * _NC + lax.axis_index("c")
    base = wid * _BPW
    pltpu.sync_copy(idx_hbm.at[pl.ds(base, _BPW)], idx_v)

    def split(g, carry):
        vec = idx_v[pl.ds(g * 16, 16)]
        q_v[pl.ds(g * 16, 16)] = vec >> 2
        return carry

    lax.fori_loop(0, _BPW // 16, split, 0)

    pltpu.async_copy(tview_hbm.at[q_v], rows2_v, sem).wait()

    def chunk(c, carry):
        sel32 = sel_v.bitcast(jnp.uint32)

        def select(g, carry2):
            pvec = idx_v[pl.ds(c * _C + g * 16, 16)] & 3
            for k in range(16):
                j = g * 16 + k
                off = pvec[k] * 32
                for h in range(2):
                    w = rows2_v[c * _C + j, pl.ds(off + h * 16, 16)]
                    sel32[j, pl.ds(h * 16, 16)] = w << 16
                    sel32[j, pl.ds(32 + h * 16, 16)] = w & _HI_MASK
            return carry2

        lax.fori_loop(0, _C // 16, select, 0)
        pltpu.sync_copy(sel_v, out_hbm.at[pl.ds(base + c * _C, _C)])
        return carry

    lax.fori_loop(0, _NCH, chunk, 0)


@jax.jit
def kernel(elements, entity_embeddings):
    idx = elements.astype(jnp.int32)
    t32 = lax.bitcast_convert_type(entity_embeddings, jnp.uint32)
    lo, hi = t32[:, :32], t32[:, 32:]
    rnd = lambda x: (x + jnp.uint32(0x7FFF) + ((x >> 16) & jnp.uint32(1))) >> 16
    packed = (rnd(hi) << 16) | rnd(lo)
    tview = packed.reshape(_V // 4, 2 * _D)

    mesh = plsc.VectorSubcoreMesh(core_axis_name="c", subcore_axis_name="s")
    f = pl.kernel(
        _gather_body,
        mesh=mesh,
        out_type=jax.ShapeDtypeStruct((_B, _D), jnp.float32),
        scratch_types=[
            pltpu.VMEM((_BPW,), jnp.int32),
            pltpu.VMEM((_BPW,), jnp.int32),
            pltpu.VMEM((_BPW, 2 * _D), jnp.uint32),
            pltpu.VMEM((_C, _D), jnp.float32),
            pltpu.SemaphoreType.DMA,
        ],
    )
    return f(idx, tview)


# speedup vs baseline: 2.1724x; 2.1724x over previous
import jax
import jax.numpy as jnp
from jax import lax
from jax.experimental import pallas as pl
from jax.experimental.pallas import tpu as pltpu
from jax.experimental.pallas import tpu_sc as plsc

_D = 64
_B = 16384

_info = plsc.get_sparse_core_info()
_NC, _NS = _info.num_cores, _info.num_subcores
_NW = _NC * _NS
_BPW = _B // _NW
_NSEM = 4
_GPS = _BPW // 16 // _NSEM


def _gather_body(idx_hbm, table_hbm, out_hbm, idx_v, rows_v, sems):
    wid = lax.axis_index("s") * _NC + lax.axis_index("c")
    base = wid * _BPW
    pltpu.sync_copy(idx_hbm.at[pl.ds(base, _BPW)], idx_v)

    @plsc.parallel_loop(0, _BPW // 16, unroll=2)
    def body(g):
        vec = idx_v[pl.ds(g * 16, 16)]
        sem = sems.at[lax.div(g, _GPS)]
        for k in range(16):
            r = vec[k]
            pltpu.make_async_copy(
                table_hbm.at[pl.ds(r, 1)],
                rows_v.at[pl.ds(g * 16 + k, 1)],
                sem,
            ).start()

    q = _GPS * 16
    for s in range(_NSEM):
        pltpu.make_async_copy(
            table_hbm.at[pl.ds(0, q)], rows_v.at[pl.ds(s * q, q)], sems.at[s]
        ).wait()

    pltpu.sync_copy(rows_v, out_hbm.at[pl.ds(base, _BPW)])


@jax.jit
def kernel(elements, entity_embeddings):
    idx = elements.astype(jnp.int32)
    mesh = plsc.VectorSubcoreMesh(core_axis_name="c", subcore_axis_name="s")
    f = pl.kernel(
        _gather_body,
        mesh=mesh,
        out_type=jax.ShapeDtypeStruct((_B, _D), jnp.float32),
        scratch_types=[
            pltpu.VMEM((_BPW,), jnp.int32),
            pltpu.VMEM((_BPW, _D), jnp.float32),
            pltpu.SemaphoreType.DMA((_NSEM,)),
        ],
    )
    return f(idx, entity_embeddings)
